# baseline (speedup 1.0000x reference)
import functools
import jax
import jax.numpy as jnp
from jax import lax
from jax.experimental import pallas as pl
from jax.experimental.pallas import tpu as pltpu
from jax.experimental.pallas import tpu_sc as plsc

D = 64
NC, NS, L = 2, 16, 16
NW = NC * NS
SCALE = 8.0
CHUNK = 320
NBUF = 2


def _make_kernel(B):
    b_per_w = B // NW
    n_chunks = b_per_w // CHUNK
    n_rounds = n_chunks // NBUF
    assert n_chunks * CHUNK == b_per_w and n_rounds * NBUF == n_chunks
    mesh = plsc.VectorSubcoreMesh(
        core_axis_name="c", subcore_axis_name="s",
        num_cores=NC, num_subcores=NS,
    )

    scratch = dict(
        idx_all=pltpu.VMEM((b_per_w,), jnp.int32),
        gbuf=[pltpu.VMEM((CHUNK, D), jnp.float32) for _ in range(NBUF)],
        sbuf=[pltpu.VMEM((CHUNK, D), jnp.float32) for _ in range(NBUF)],
        gsem=[pltpu.SemaphoreType.DMA for _ in range(NBUF)],
        ssem=[pltpu.SemaphoreType.DMA for _ in range(NBUF)],
    )

    @functools.partial(
        pl.kernel,
        mesh=mesh,
        compiler_params=pltpu.CompilerParams(use_tc_tiling_on_sc=False),
        out_type=jax.ShapeDtypeStruct((B, D), jnp.float32),
        scratch_types=scratch,
    )
    def k(x_hbm, table_hbm, out_hbm, idx_all, gbuf, sbuf, gsem, ssem):
        wid = lax.axis_index("s") * NC + lax.axis_index("c")
        base = wid * b_per_w

        pltpu.sync_copy(x_hbm.at[pl.ds(base, b_per_w)], idx_all)

        def issue_gather(c, b):
            pltpu.async_copy(
                table_hbm.at[idx_all.at[pl.ds(c * CHUNK, CHUNK)]],
                gbuf[b], gsem[b])

        def issue_store(c, b):
            pltpu.async_copy(
                sbuf[b], out_hbm.at[pl.ds(base + c * CHUNK, CHUNK)], ssem[b])

        def wait_gather(b):
            pltpu.make_async_copy(table_hbm.at[idx_all.at[pl.ds(0, CHUNK)]],
                                  gbuf[b], gsem[b]).wait()

        def wait_store(b):
            pltpu.make_async_copy(sbuf[b], out_hbm.at[pl.ds(0, CHUNK)],
                                  ssem[b]).wait()

        def scale(b):
            def row_body(r, carry):
                for j in range(D // L):
                    sl = pl.ds(j * L, L)
                    sbuf[b][r, sl] = gbuf[b][r, sl] * SCALE
                return carry
            lax.fori_loop(0, CHUNK, row_body, 0, unroll=8)

        for b in range(NBUF):
            issue_gather(b, b)

        for b in range(NBUF):
            wait_gather(b)
            scale(b)
            issue_gather(NBUF + b, b)
            issue_store(b, b)

        def round_body(r, carry):
            c0 = r * NBUF
            for b in range(NBUF):
                c = c0 + b
                wait_gather(b)
                wait_store(b)
                scale(b)
                issue_gather(c + NBUF, b)
                issue_store(c, b)
            return carry
        lax.fori_loop(1, n_rounds - 1, round_body, 0)

        for b in range(NBUF):
            c = (n_rounds - 1) * NBUF + b
            wait_gather(b)
            wait_store(b)
            scale(b)
            issue_store(c, b)
        for b in range(NBUF):
            wait_store(b)

    return k


def kernel(x, table):
    B = x.shape[0] * x.shape[1]
    xf = x.reshape(B).astype(jnp.int32)
    out = _make_kernel(B)(xf, table)
    return out.reshape(x.shape + (D,))
